# Initial kernel scaffold; baseline (speedup 1.0000x reference)
import jax
import jax.numpy as jnp
from jax.experimental import pallas as pl


def kernel(x, table):
    raise NotImplementedError("write your pallas kernel here")


# speedup vs baseline: 1.0954x; 1.0954x over previous
import functools

import jax
import jax.numpy as jnp
from jax import lax
from jax.experimental import pallas as pl
from jax.experimental.pallas import tpu as pltpu
from jax.experimental.pallas import tpu_sc as plsc


def _gather_kernel(B, V, D, n_workers, num_cores, chunk):
    b_per_w = B // n_workers
    n_chunks = b_per_w // chunk
    mesh = plsc.VectorSubcoreMesh(core_axis_name="c", subcore_axis_name="s")

    @functools.partial(
        pl.kernel,
        mesh=mesh,
        out_type=jax.ShapeDtypeStruct((B, D), jnp.float32),
        scratch_types=[
            pltpu.VMEM((chunk,), jnp.int32),
            pltpu.VMEM((chunk, D), jnp.float32),
            pltpu.SemaphoreType.DMA,
        ],
        compiler_params=pltpu.CompilerParams(use_tc_tiling_on_sc=False),
    )
    def k(idx_hbm, table_hbm, out_hbm, idx_v, rows_v, sem):
        wid = lax.axis_index("s") * num_cores + lax.axis_index("c")
        wbase = wid * b_per_w

        def body(c, carry):
            base = wbase + c * chunk
            pltpu.sync_copy(idx_hbm.at[pl.ds(base, chunk)], idx_v)
            pltpu.async_copy(table_hbm.at[idx_v], rows_v, sem).wait()
            pltpu.sync_copy(rows_v, out_hbm.at[pl.ds(base, chunk)])
            return carry

        lax.fori_loop(0, n_chunks, body, 0)

    return k


def kernel(x, table):
    B0, B1 = x.shape
    V, D = table.shape
    B = B0 * B1
    idx = x.reshape(B)

    info = plsc.get_sparse_core_info()
    n_workers = info.num_cores * info.num_subcores
    k = _gather_kernel(B, V, D, n_workers, info.num_cores, chunk=1024)
    out = k(idx, table)
    return out.reshape(B0, B1, D)


# speedup vs baseline: 1.1133x; 1.0163x over previous
import functools

import jax
import jax.numpy as jnp
from jax import lax
from jax.experimental import pallas as pl
from jax.experimental.pallas import tpu as pltpu
from jax.experimental.pallas import tpu_sc as plsc

_CHUNK = 1024
_NBUF = 3
_DEPTH = 2


def _gather_kernel(B, D, n_workers, num_cores):
    b_per_w = B // n_workers
    n_chunks = b_per_w // _CHUNK
    mesh = plsc.VectorSubcoreMesh(core_axis_name="c", subcore_axis_name="s")

    @functools.partial(
        pl.kernel,
        mesh=mesh,
        out_type=jax.ShapeDtypeStruct((B, D), jnp.float32),
        scratch_types=[
            pltpu.VMEM((_NBUF, _CHUNK), jnp.int32),
            pltpu.VMEM((_NBUF, _CHUNK, D), jnp.float32),
        ]
        + [pltpu.SemaphoreType.DMA] * (3 * _NBUF),
        compiler_params=pltpu.CompilerParams(use_tc_tiling_on_sc=False),
    )
    def k(idx_hbm, table_hbm, out_hbm, idx_v, rows_v, *sems):
        isem = sems[:_NBUF]
        gsem = sems[_NBUF : 2 * _NBUF]
        ssem = sems[2 * _NBUF :]
        wid = lax.axis_index("s") * num_cores + lax.axis_index("c")
        wbase = wid * b_per_w

        h_idx = [None] * n_chunks
        h_g = [None] * n_chunks
        h_s = [None] * n_chunks

        def start_idx(c):
            b = c % _NBUF
            h_idx[c] = pltpu.async_copy(
                idx_hbm.at[pl.ds(wbase + c * _CHUNK, _CHUNK)], idx_v.at[b], isem[b]
            )

        def start_gather(c):
            b = c % _NBUF
            h_g[c] = pltpu.async_copy(
                table_hbm.at[idx_v.at[b]], rows_v.at[b], gsem[b]
            )

        def start_store(c):
            b = c % _NBUF
            h_s[c] = pltpu.async_copy(
                rows_v.at[b], out_hbm.at[pl.ds(wbase + c * _CHUNK, _CHUNK)], ssem[b]
            )

        for c in range(_NBUF):
            start_idx(c)
        for c in range(n_chunks):
            if c >= _NBUF:
                h_s[c - _NBUF].wait()
            h_idx[c].wait()
            start_gather(c)
            d = c - _DEPTH
            if d >= 0:
                h_g[d].wait()
                start_store(d)
                if d + _NBUF < n_chunks:
                    start_idx(d + _NBUF)
        for d in range(n_chunks - _DEPTH, n_chunks):
            h_g[d].wait()
            start_store(d)
        for c in range(n_chunks - _NBUF, n_chunks):
            h_s[c].wait()

    return k


def kernel(x, table):
    B0, B1 = x.shape
    V, D = table.shape
    B = B0 * B1
    idx = x.reshape(B)

    info = plsc.get_sparse_core_info()
    n_workers = info.num_cores * info.num_subcores
    k = _gather_kernel(B, D, n_workers, info.num_cores)
    out = k(idx, table)
    return out.reshape(B0, B1, D)


# speedup vs baseline: 1.4898x; 1.3382x over previous
import functools

import jax
import jax.numpy as jnp
from jax import lax
from jax.experimental import pallas as pl
from jax.experimental.pallas import tpu as pltpu
from jax.experimental.pallas import tpu_sc as plsc

_UNIT = 512


def _build(B0, B1, V, D, n_workers, num_cores):
    ipj = B0 // _UNIT
    n_units = ipj * B1
    upw = n_units // n_workers
    mesh = plsc.VectorSubcoreMesh(core_axis_name="c", subcore_axis_name="s")
    n_sub = _UNIT // 16

    @functools.partial(
        pl.kernel,
        mesh=mesh,
        out_type=jax.ShapeDtypeStruct((B1 * D, B0), jnp.float32),
        scratch_types=[
            pltpu.VMEM((2, _UNIT), jnp.int32),
            pltpu.VMEM((2, _UNIT, D), jnp.float32),
            pltpu.VMEM((2, D, _UNIT), jnp.float32),
        ]
        + [pltpu.SemaphoreType.DMA] * 6,
        compiler_params=pltpu.CompilerParams(
            use_tc_tiling_on_sc=False, needs_layout_passes=False
        ),
    )
    def k(xt_hbm, table_hbm, out_hbm, idx_v, rows_v, tr_v, *sems):
        isem = sems[0:2]
        gsem = sems[2:4]
        ssem = sems[4:6]
        wid = lax.axis_index("s") * num_cores + lax.axis_index("c")
        u0 = wid * upw

        def unit_pos(u):
            uu = u0 + u
            j = uu // ipj
            i0 = (uu % ipj) * _UNIT
            return j, i0

        def start_idx(u, b):
            j, i0 = unit_pos(u)
            pltpu.async_copy(
                xt_hbm.at[pl.ds(j * B0 + i0, _UNIT)], idx_v.at[b], isem[b]
            )

        def wait_idx(b):
            pltpu.make_async_copy(
                xt_hbm.at[pl.ds(0, _UNIT)], idx_v.at[b], isem[b]
            ).wait()

        def start_gather(b):
            pltpu.async_copy(table_hbm.at[idx_v.at[b]], rows_v.at[b], gsem[b])

        def wait_gather(b):
            pltpu.make_async_copy(
                table_hbm.at[pl.ds(0, _UNIT)], rows_v.at[b], gsem[b]
            ).wait()

        def start_store(u, b):
            j, i0 = unit_pos(u)
            pltpu.async_copy(
                tr_v.at[b],
                out_hbm.at[pl.ds(j * D, D), pl.ds(i0, _UNIT)],
                ssem[b],
            )

        def wait_store(b):
            pltpu.make_async_copy(
                out_hbm.at[pl.ds(0, D), pl.ds(0, _UNIT)], tr_v.at[b], ssem[b]
            ).wait()

        iota = lax.iota(jnp.int32, 16)
        zeros16 = jnp.zeros((16,), jnp.int32)

        def transpose(b):
            rowis = [iota + (s * 16) for s in range(n_sub)]

            def cbody(c, carry):
                csplat = zeros16 + c
                for s in range(n_sub):
                    vals = plsc.load_gather(rows_v.at[b], [rowis[s], csplat])
                    plsc.store_scatter(tr_v.at[b], [csplat, rowis[s]], vals)
                return carry

            lax.fori_loop(0, D, cbody, 0)

        def finish_dyn(u, b, guard):
            wait_gather(b)
            start_idx(u + 2, b)

            @pl.when(guard)
            def _():
                wait_store(b)

            transpose(b)
            start_store(u, b)

        start_idx(0, 0)
        start_idx(1, 1)
        wait_idx(0)
        start_gather(0)

        def body(g, carry):
            u = 2 * g
            wait_idx(1)
            start_gather(1)
            finish_dyn(u, 0, g >= 1)
            wait_idx(0)
            start_gather(0)
            finish_dyn(u + 1, 1, g >= 1)
            return carry

        lax.fori_loop(0, (upw - 2) // 2, body, 0)

        u = upw - 2
        wait_idx(1)
        start_gather(1)
        wait_gather(0)
        wait_store(0)
        transpose(0)
        start_store(u, 0)
        wait_gather(1)
        wait_store(1)
        transpose(1)
        start_store(u + 1, 1)
        wait_store(0)
        wait_store(1)

    return k


def kernel(x, table):
    B0, B1 = x.shape
    V, D = table.shape

    xt = jnp.swapaxes(x, 0, 1).reshape(B1 * B0)

    info = plsc.get_sparse_core_info()
    n_workers = info.num_cores * info.num_subcores
    k = _build(B0, B1, V, D, n_workers, info.num_cores)
    out2d = k(xt, table)
    return out2d.reshape(B1, D, B0).transpose(2, 0, 1)


# speedup vs baseline: 1.6630x; 1.1163x over previous
import functools

import jax
import jax.numpy as jnp
from jax import lax
from jax.experimental import pallas as pl
from jax.experimental.pallas import tpu as pltpu
from jax.experimental.pallas import tpu_sc as plsc

_UNIT = 512


def _build(B0, B1, V, D, n_workers, num_cores):
    ipj = B0 // _UNIT
    n_units = ipj * B1
    upw = n_units // n_workers
    mesh = plsc.VectorSubcoreMesh(core_axis_name="c", subcore_axis_name="s")
    n_sub = _UNIT // 16

    @functools.partial(
        pl.kernel,
        mesh=mesh,
        out_type=jax.ShapeDtypeStruct((B1 * D, B0), jnp.float32),
        scratch_types=[
            pltpu.VMEM((2, _UNIT), jnp.int32),
            pltpu.VMEM((2, _UNIT, D), jnp.float32),
            pltpu.VMEM((2, D, _UNIT), jnp.float32),
        ]
        + [pltpu.SemaphoreType.DMA] * 6,
        compiler_params=pltpu.CompilerParams(
            use_tc_tiling_on_sc=False, needs_layout_passes=False
        ),
    )
    def k(xt_hbm, table_hbm, out_hbm, idx_v, rows_v, tr_v, *sems):
        isem = sems[0:2]
        gsem = sems[2:4]
        ssem = sems[4:6]
        wid = lax.axis_index("s") * num_cores + lax.axis_index("c")
        u0 = wid * upw

        def unit_pos(u):
            uu = u0 + u
            j = uu // ipj
            i0 = (uu % ipj) * _UNIT
            return j, i0

        def start_idx(u, b):
            j, i0 = unit_pos(u)
            pltpu.async_copy(
                xt_hbm.at[pl.ds(j * B0 + i0, _UNIT)], idx_v.at[b], isem[b]
            )

        def wait_idx(b):
            pltpu.make_async_copy(
                xt_hbm.at[pl.ds(0, _UNIT)], idx_v.at[b], isem[b]
            ).wait()

        def start_gather(b):
            pltpu.async_copy(table_hbm.at[idx_v.at[b]], rows_v.at[b], gsem[b])

        def wait_gather(b):
            pltpu.make_async_copy(
                table_hbm.at[pl.ds(0, _UNIT)], rows_v.at[b], gsem[b]
            ).wait()

        def start_store(u, b):
            j, i0 = unit_pos(u)
            pltpu.async_copy(
                tr_v.at[b],
                out_hbm.at[pl.ds(j * D, D), pl.ds(i0, _UNIT)],
                ssem[b],
            )

        def wait_store(b):
            pltpu.make_async_copy(
                out_hbm.at[pl.ds(0, D), pl.ds(0, _UNIT)], tr_v.at[b], ssem[b]
            ).wait()

        iota = lax.iota(jnp.int32, 16)
        zeros16 = jnp.zeros((16,), jnp.int32)
        cvecs = [iota + (h * 16) for h in range(D // 16)]

        def transpose(b):
            def rbody(r4, carry):
                r = r4 * 4
                for dr in range(4):
                    rr = r + dr
                    rs = zeros16 + rr
                    for h in range(D // 16):
                        v = rows_v[b, rr, pl.ds(h * 16, 16)]
                        plsc.store_scatter(tr_v.at[b], [cvecs[h], rs], v)
                return carry

            lax.fori_loop(0, _UNIT // 4, rbody, 0)

        def finish_dyn(u, b, guard):
            wait_gather(b)
            start_idx(u + 2, b)

            @pl.when(guard)
            def _():
                wait_store(b)

            transpose(b)
            start_store(u, b)

        start_idx(0, 0)
        start_idx(1, 1)
        wait_idx(0)
        start_gather(0)

        def body(g, carry):
            u = 2 * g
            wait_idx(1)
            start_gather(1)
            finish_dyn(u, 0, g >= 1)
            wait_idx(0)
            start_gather(0)
            finish_dyn(u + 1, 1, g >= 1)
            return carry

        lax.fori_loop(0, (upw - 2) // 2, body, 0)

        u = upw - 2
        wait_idx(1)
        start_gather(1)
        wait_gather(0)
        wait_store(0)
        transpose(0)
        start_store(u, 0)
        wait_gather(1)
        wait_store(1)
        transpose(1)
        start_store(u + 1, 1)
        wait_store(0)
        wait_store(1)

    return k


def kernel(x, table):
    B0, B1 = x.shape
    V, D = table.shape

    xt = jnp.swapaxes(x, 0, 1).reshape(B1 * B0)

    info = plsc.get_sparse_core_info()
    n_workers = info.num_cores * info.num_subcores
    k = _build(B0, B1, V, D, n_workers, info.num_cores)
    out2d = k(xt, table)
    return out2d.reshape(B1, D, B0).transpose(2, 0, 1)


# speedup vs baseline: 2.1874x; 1.3153x over previous
import functools

import jax
import jax.numpy as jnp
from jax import lax
from jax.experimental import pallas as pl
from jax.experimental.pallas import tpu as pltpu
from jax.experimental.pallas import tpu_sc as plsc

_UNIT = 512


def _build(B0, B1, V, D, n_workers, num_cores):
    mesh = plsc.VectorSubcoreMesh(core_axis_name="c", subcore_axis_name="s")
    n_sub = _UNIT // 16
    assert B0 % (_UNIT * n_workers) == 0 or B0 == _UNIT * n_workers
    assert B1 % 2 == 0

    @functools.partial(
        pl.kernel,
        mesh=mesh,
        out_type=jax.ShapeDtypeStruct((B1 * D, B0), jnp.float32),
        scratch_types=[
            pltpu.VMEM((_UNIT, B1), jnp.int32),
            pltpu.VMEM((2, _UNIT), jnp.int32),
            pltpu.VMEM((2, _UNIT, D), jnp.float32),
            pltpu.VMEM((2, D, _UNIT), jnp.float32),
        ]
        + [pltpu.SemaphoreType.DMA] * 5,
        compiler_params=pltpu.CompilerParams(
            use_tc_tiling_on_sc=False, needs_layout_passes=False
        ),
    )
    def k(x_hbm, table_hbm, out_hbm, xblk_v, idx_v, rows_v, tr_v, *sems):
        xsem = sems[0]
        gsem = sems[1:3]
        ssem = sems[3:5]
        wid = lax.axis_index("s") * num_cores + lax.axis_index("c")
        i0 = wid * _UNIT

        iota = lax.iota(jnp.int32, 16)
        zeros16 = jnp.zeros((16,), jnp.int32)
        cvecs = [iota + (h * 16) for h in range(D // 16)]
        rowvecs = [iota + (s * 16) for s in range(n_sub)]

        def build_idx(j, b):
            jsplat = zeros16 + j
            for s in range(n_sub):
                v = plsc.load_gather(xblk_v, [rowvecs[s], jsplat])
                idx_v[b, pl.ds(s * 16, 16)] = v

        def start_gather(b):
            pltpu.async_copy(table_hbm.at[idx_v.at[b]], rows_v.at[b], gsem[b])

        def wait_gather(b):
            pltpu.make_async_copy(
                table_hbm.at[pl.ds(0, _UNIT)], rows_v.at[b], gsem[b]
            ).wait()

        def start_store(j, b):
            pltpu.async_copy(
                tr_v.at[b],
                out_hbm.at[pl.ds(j * D, D), pl.ds(i0, _UNIT)],
                ssem[b],
            )

        def wait_store(b):
            pltpu.make_async_copy(
                out_hbm.at[pl.ds(0, D), pl.ds(0, _UNIT)], tr_v.at[b], ssem[b]
            ).wait()

        def transpose(b):
            def dbody(b4, carry):
                base = b4 * 4
                for db in range(4):
                    bs = zeros16 + (base + db)
                    for h in range(D // 16):
                        rvec = (bs + cvecs[h]) & (_UNIT - 1)
                        v = plsc.load_gather(rows_v.at[b], [rvec, cvecs[h]])
                        plsc.store_scatter(tr_v.at[b], [cvecs[h], rvec], v)
                return carry

            lax.fori_loop(0, _UNIT // 4, dbody, 0)

        def finish(j, b, guard):
            wait_gather(b)

            @pl.when(guard)
            def _():
                wait_store(b)

            transpose(b)
            start_store(j, b)

        pltpu.async_copy(x_hbm.at[pl.ds(i0, _UNIT), :], xblk_v, xsem)
        pltpu.make_async_copy(
            x_hbm.at[pl.ds(0, _UNIT), :], xblk_v, xsem
        ).wait()
        build_idx(0, 0)
        start_gather(0)

        def body(g, carry):
            j = 2 * g
            build_idx(j + 1, 1)
            start_gather(1)
            finish(j, 0, g >= 1)
            build_idx(j + 2, 0)
            start_gather(0)
            finish(j + 1, 1, g >= 1)
            return carry

        lax.fori_loop(0, (B1 - 2) // 2, body, 0)

        build_idx(B1 - 1, 1)
        start_gather(1)
        wait_gather(0)
        wait_store(0)
        transpose(0)
        start_store(B1 - 2, 0)
        wait_gather(1)
        wait_store(1)
        transpose(1)
        start_store(B1 - 1, 1)
        wait_store(0)
        wait_store(1)

    return k


def kernel(x, table):
    B0, B1 = x.shape
    V, D = table.shape

    info = plsc.get_sparse_core_info()
    n_workers = info.num_cores * info.num_subcores
    k = _build(B0, B1, V, D, n_workers, info.num_cores)
    out2d = k(x, table)
    return out2d.reshape(B1, D, B0).transpose(2, 0, 1)


# speedup vs baseline: 2.2172x; 1.0137x over previous
import functools

import jax
import jax.numpy as jnp
from jax import lax
from jax.experimental import pallas as pl
from jax.experimental.pallas import tpu as pltpu
from jax.experimental.pallas import tpu_sc as plsc

_UNIT = 512


def _build(B0, B1, V, D, n_workers, num_cores):
    mesh = plsc.VectorSubcoreMesh(core_axis_name="c", subcore_axis_name="s")
    n_sub = _UNIT // 16
    assert B0 % (_UNIT * n_workers) == 0 or B0 == _UNIT * n_workers
    assert B1 % 2 == 0

    @functools.partial(
        pl.kernel,
        mesh=mesh,
        out_type=jax.ShapeDtypeStruct((B1 * D, B0), jnp.float32),
        scratch_types=[
            pltpu.VMEM((_UNIT, B1), jnp.int32),
            pltpu.VMEM((2, _UNIT), jnp.int32),
            pltpu.VMEM((2, _UNIT, D), jnp.float32),
            pltpu.VMEM((2, D, _UNIT), jnp.float32),
        ]
        + [pltpu.SemaphoreType.DMA] * 5,
        compiler_params=pltpu.CompilerParams(
            use_tc_tiling_on_sc=False, needs_layout_passes=False
        ),
    )
    def k(x_hbm, table_hbm, out_hbm, xblk_v, idx_v, rows_v, tr_v, *sems):
        xsem = sems[0]
        gsem = sems[1:3]
        ssem = sems[3:5]
        wid = lax.axis_index("s") * num_cores + lax.axis_index("c")
        i0 = wid * _UNIT

        iota = lax.iota(jnp.int32, 16)
        zeros16 = jnp.zeros((16,), jnp.int32)
        cvecs = [iota + (h * 16) for h in range(D // 16)]
        rowvecs = [iota + (s * 16) for s in range(n_sub)]

        def build_idx(j, b):
            jsplat = zeros16 + j
            for s in range(n_sub):
                v = plsc.load_gather(xblk_v, [rowvecs[s], jsplat])
                idx_v[b, pl.ds(s * 16, 16)] = v * 4

        def start_gather(b):
            pltpu.async_copy(table_hbm.at[idx_v.at[b]], rows_v.at[b], gsem[b])

        def wait_gather(b):
            pltpu.make_async_copy(
                table_hbm.at[pl.ds(0, _UNIT)], rows_v.at[b], gsem[b]
            ).wait()

        def start_store(j, b):
            pltpu.async_copy(
                tr_v.at[b],
                out_hbm.at[pl.ds(j * D, D), pl.ds(i0, _UNIT)],
                ssem[b],
            )

        def wait_store(b):
            pltpu.make_async_copy(
                out_hbm.at[pl.ds(0, D), pl.ds(0, _UNIT)], tr_v.at[b], ssem[b]
            ).wait()

        def transpose(b):
            def dbody(b4, carry):
                base = b4 * 4
                for db in range(4):
                    bs = zeros16 + (base + db)
                    for h in range(D // 16):
                        rvec = (bs + cvecs[h]) & (_UNIT - 1)
                        v = plsc.load_gather(rows_v.at[b], [rvec, cvecs[h]])
                        plsc.store_scatter(tr_v.at[b], [cvecs[h], rvec], v)
                return carry

            lax.fori_loop(0, _UNIT // 4, dbody, 0)

        def finish(j, b, guard):
            wait_gather(b)

            @pl.when(guard)
            def _():
                wait_store(b)

            transpose(b)
            start_store(j, b)

        pltpu.async_copy(x_hbm.at[pl.ds(i0, _UNIT), :], xblk_v, xsem)
        pltpu.make_async_copy(
            x_hbm.at[pl.ds(0, _UNIT), :], xblk_v, xsem
        ).wait()
        build_idx(0, 0)
        start_gather(0)

        def body(g, carry):
            j = 2 * g
            build_idx(j + 1, 1)
            start_gather(1)
            finish(j, 0, g >= 1)
            build_idx(j + 2, 0)
            start_gather(0)
            finish(j + 1, 1, g >= 1)
            return carry

        lax.fori_loop(0, (B1 - 2) // 2, body, 0)

        build_idx(B1 - 1, 1)
        start_gather(1)
        wait_gather(0)
        wait_store(0)
        transpose(0)
        start_store(B1 - 2, 0)
        wait_gather(1)
        wait_store(1)
        transpose(1)
        start_store(B1 - 1, 1)
        wait_store(0)
        wait_store(1)

    return k


def kernel(x, table):
    B0, B1 = x.shape
    V, D = table.shape

    info = plsc.get_sparse_core_info()
    n_workers = info.num_cores * info.num_subcores
    table4 = jnp.pad(table, ((0, 0), (0, 128 - D))).reshape(V * (128 // D), D)
    k = _build(B0, B1, V, D, n_workers, info.num_cores)
    out2d = k(x, table4)
    return out2d.reshape(B1, D, B0).transpose(2, 0, 1)
